# baseline (device time: 12929 ns/iter reference)
import jax
import jax.numpy as jnp
from jax import lax
from jax.experimental import pallas as pl
from jax.experimental.pallas import tpu as pltpu

N_DEV = 4
CH = 8


def kernel(x):
    m_per, n = x.shape
    assert m_per % (CH * 8) == 0
    c_m = m_per // CH

    def body(
        x_hbm,
        out_ref,
        bufs,
        send_buf,
        comm_ref,
        copy_sems,
        send_sems,
        recv_sems,
    ):
        my_pos = lax.axis_index("i")

        barrier_sem = pltpu.get_barrier_semaphore()
        for nbr in (
            lax.rem(my_pos + 1, N_DEV),
            lax.rem(my_pos + N_DEV - 1, N_DEV),
        ):
            pl.semaphore_signal(
                barrier_sem,
                inc=1,
                device_id=(nbr,),
                device_id_type=pl.DeviceIdType.MESH,
            )

        copies = []
        for c in range(CH):
            cp = pltpu.make_async_copy(
                x_hbm.at[pl.ds(c * c_m, c_m)], bufs.at[c], copy_sems.at[c]
            )
            cp.start()
            copies.append(cp)

        acc = None
        for c in range(CH):
            copies[c].wait()
            p = jnp.max(bufs[c].reshape(c_m // 8, 8, n), axis=0)
            acc = p if acc is None else jnp.maximum(acc, p)
        send_buf[...] = jnp.max(acc, axis=0, keepdims=True)

        pl.semaphore_wait(barrier_sem, 2)

        rdmas = []
        for d in range(1, N_DEV):
            rdma = pltpu.make_async_remote_copy(
                src_ref=send_buf,
                dst_ref=comm_ref.at[d - 1],
                send_sem=send_sems.at[d - 1],
                recv_sem=recv_sems.at[d - 1],
                device_id=(lax.rem(my_pos + d, N_DEV),),
                device_id_type=pl.DeviceIdType.MESH,
            )
            rdma.start()
            rdmas.append(rdma)
        res = send_buf[...]
        for d in range(1, N_DEV):
            rdmas[d - 1].wait_recv()
            res = jnp.maximum(res, comm_ref[d - 1])
        out_ref[...] = res
        for d in range(1, N_DEV):
            rdmas[d - 1].wait_send()

    return pl.pallas_call(
        body,
        out_shape=jax.ShapeDtypeStruct((1, n), x.dtype),
        in_specs=[pl.BlockSpec(memory_space=pltpu.MemorySpace.HBM)],
        out_specs=pl.BlockSpec(memory_space=pltpu.VMEM),
        scratch_shapes=[
            pltpu.VMEM((CH, c_m, n), x.dtype),
            pltpu.VMEM((1, n), x.dtype),
            pltpu.VMEM((N_DEV - 1, 1, n), x.dtype),
            pltpu.SemaphoreType.DMA((CH,)),
            pltpu.SemaphoreType.DMA((N_DEV - 1,)),
            pltpu.SemaphoreType.DMA((N_DEV - 1,)),
        ],
        compiler_params=pltpu.CompilerParams(collective_id=0),
    )(x)
